# baseline (speedup 1.0000x reference)
import functools

import jax
import jax.numpy as jnp
from jax import lax
from jax.experimental import pallas as pl
from jax.experimental.pallas import tpu as pltpu
from jax.experimental.pallas import tpu_sc as plsc

N_NODES = 10000
D = 128
EXPERTS = 3
TEMP = 101.0

NC = 2
NS = 16
NW = NC * NS
L = 16

CHUNK = 128
RING = 4
NPAD = 10240
RPT = NPAD // NS


def _mesh():
    return plsc.VectorSubcoreMesh(
        core_axis_name="c", subcore_axis_name="s", num_cores=NC, num_subcores=NS
    )


def _chunk_plan(e):
    nchunks = -(-e // CHUNK)
    base = nchunks // NW
    extra = nchunks % NW
    return base, extra, e % CHUNK


def _wid_start(wid, base, extra):
    return wid * base + jnp.minimum(wid, extra)


def _deg_call(ei, base, extra, tail):
    assert tail == 0, "deg kernel assumes CHUNK-aligned edge count"

    pairs = base // 2
    assert base % 2 == 0, "deg loop assumes even base chunk count"

    @functools.partial(
        pl.kernel,
        mesh=_mesh(),
        out_type=jax.ShapeDtypeStruct((NC, NPAD), jnp.float32),
        scratch_types=[
            pltpu.VMEM((2, CHUNK), jnp.int32),
            pltpu.VMEM((CHUNK,), jnp.float32),
            pltpu.VMEM((RPT,), jnp.float32),
            pltpu.VMEM_SHARED((NPAD,), jnp.float32),
            pltpu.SemaphoreType.DMA,
            pltpu.SemaphoreType.DMA,
        ],
    )
    def k(ei_hbm, out_hbm, idxv, ones_v, zbuf, deg_sh, ses, sos):
        cid = lax.axis_index("c")
        sid = lax.axis_index("s")
        wid = sid * NC + cid
        start = _wid_start(wid, base, extra) * CHUNK
        n = base + jnp.where(wid < extra, 1, 0)

        def fill_ones(i, carry):
            ones_v[pl.ds(i * L, L)] = jnp.ones((L,), jnp.float32)
            return carry

        lax.fori_loop(0, CHUNK // L, fill_ones, 0)

        def fill_zero(i, carry):
            zbuf[pl.ds(i * L, L)] = jnp.zeros((L,), jnp.float32)
            return carry

        lax.fori_loop(0, RPT // L, fill_zero, 0)

        pltpu.sync_copy(zbuf, deg_sh.at[pl.ds(sid * RPT, RPT)])

        def ipf(c, slot, sem):
            pltpu.async_copy(
                ei_hbm.at[1, pl.ds(start + c * CHUNK, CHUNK)], idxv.at[slot], sem
            )

        def iwait(sem):
            pltpu.make_async_copy(
                ei_hbm.at[1, pl.ds(0, CHUNK)], idxv.at[0], sem
            ).wait()

        ipf(0, 0, ses)
        ipf(1, 1, sos)
        plsc.subcore_barrier()

        def body(i, carry):
            c = 2 * i
            iwait(ses)
            pltpu.sync_copy(ones_v, deg_sh.at[idxv.at[0]], add=True)

            @pl.when(c + 2 < n)
            def _():
                ipf(c + 2, 0, ses)

            iwait(sos)
            pltpu.sync_copy(ones_v, deg_sh.at[idxv.at[1]], add=True)

            @pl.when(c + 3 < n)
            def _():
                ipf(c + 3, 1, sos)

            return carry

        lax.fori_loop(0, pairs, body, 0)
        if extra:
            @pl.when(wid < extra)
            def _():
                iwait(ses)
                pltpu.sync_copy(ones_v, deg_sh.at[idxv.at[0]], add=True)
        plsc.subcore_barrier()
        pltpu.sync_copy(
            deg_sh.at[pl.ds(sid * RPT, RPT)],
            out_hbm.at[cid, pl.ds(sid * RPT, RPT)],
        )

    return k(ei)


def _agg_call(xs, ei, base, extra, tail):
    assert tail == 0, "agg kernel assumes CHUNK-aligned edge count"
    pairs = base // 2
    odd = base % 2

    @functools.partial(
        pl.kernel,
        mesh=_mesh(),
        out_type=jax.ShapeDtypeStruct((NC, NPAD, D), jnp.float32),
        scratch_types=[
            pltpu.VMEM((RING, CHUNK), jnp.int32),
            pltpu.VMEM((RING, CHUNK), jnp.int32),
            pltpu.VMEM((CHUNK, D), jnp.float32),
            pltpu.VMEM((CHUNK, D), jnp.float32),
            pltpu.VMEM_SHARED((NPAD, D), jnp.float32),
            pltpu.SemaphoreType.DMA,
            pltpu.SemaphoreType.DMA,
            pltpu.SemaphoreType.DMA,
            pltpu.SemaphoreType.DMA,
        ],
    )
    def k(xs_hbm, ei_hbm, out_hbm, sxv, dxv, ra, rb, agg_sh, sie, sio, sa, sb):
        cid = lax.axis_index("c")
        sid = lax.axis_index("s")
        wid = sid * NC + cid
        start = _wid_start(wid, base, extra) * CHUNK
        n = base + jnp.where(wid < extra, 1, 0)

        def zfill(i, carry):
            rb[lax.div(i, D // L), pl.ds(lax.rem(i, D // L) * L, L)] = jnp.zeros(
                (L,), jnp.float32
            )
            return carry

        lax.fori_loop(0, CHUNK * (D // L), zfill, 0)
        nz = -(-RPT // CHUNK)
        for kk in range(nz):
            rows = min(CHUNK, RPT - kk * CHUNK)
            pltpu.sync_copy(
                rb.at[pl.ds(0, rows)],
                agg_sh.at[pl.ds(sid * RPT + kk * CHUNK, rows)],
            )

        def ipf(c, slot, sem):
            off = start + c * CHUNK
            pltpu.async_copy(ei_hbm.at[0, pl.ds(off, CHUNK)], sxv.at[slot], sem)
            pltpu.async_copy(ei_hbm.at[1, pl.ds(off, CHUNK)], dxv.at[slot], sem)

        def iwait(sem):
            pltpu.make_async_copy(ei_hbm.at[0, pl.ds(0, CHUNK)], sxv.at[0], sem).wait()
            pltpu.make_async_copy(ei_hbm.at[1, pl.ds(0, CHUNK)], dxv.at[0], sem).wait()

        def gwait(buf, sem):
            pltpu.make_async_copy(xs_hbm.at[sxv.at[0]], buf, sem).wait()

        ipf(0, 0, sie)
        iwait(sie)
        ipf(1, 1, sio)
        ipf(2, 2, sie)
        plsc.subcore_barrier()
        pltpu.async_copy(xs_hbm.at[sxv.at[0]], ra, sa)

        def body(i, carry):
            c = 2 * i
            r0 = lax.rem(c, RING)
            r1 = lax.rem(c + 1, RING)
            r2 = lax.rem(c + 2, RING)
            r3 = lax.rem(c + 3, RING)
            iwait(sio)
            pltpu.async_copy(xs_hbm.at[sxv.at[r1]], rb, sb)

            @pl.when(c + 3 < n)
            def _():
                ipf(c + 3, r3, sio)

            gwait(ra, sa)
            pltpu.sync_copy(ra, agg_sh.at[dxv.at[r0]], add=True)

            @pl.when(c + 2 < n)
            def _():
                iwait(sie)
                pltpu.async_copy(xs_hbm.at[sxv.at[r2]], ra, sa)

                @pl.when(c + 4 < n)
                def _():
                    ipf(c + 4, lax.rem(c + 4, RING), sie)

            gwait(rb, sb)
            pltpu.sync_copy(rb, agg_sh.at[dxv.at[r1]], add=True)
            return carry

        lax.fori_loop(0, pairs, body, 0)
        if odd:
            rz = (base - 1) % RING
            buf, sem = (ra, sa) if (base - 1) % 2 == 0 else (rb, sb)
            gwait(buf, sem)
            pltpu.sync_copy(buf, agg_sh.at[dxv.at[rz]], add=True)
        if extra:
            @pl.when(wid < extra)
            def _():
                rz = base % RING
                buf, sem = (ra, sa) if base % 2 == 0 else (rb, sb)
                gwait(buf, sem)
                pltpu.sync_copy(buf, agg_sh.at[dxv.at[rz]], add=True)
        plsc.subcore_barrier()
        pltpu.sync_copy(
            agg_sh.at[pl.ds(sid * RPT, RPT)],
            out_hbm.at[cid, pl.ds(sid * RPT, RPT)],
        )

    return k(xs, ei)


def _prep_call(deg_parts, x, gf, Wg):
    def body(degp_ref, x_ref, gf_ref, wg_ref, xs_ref, dsi_ref, gate_ref):
        deg = degp_ref[0] + degp_ref[1] + 1.0
        dsi = lax.rsqrt(deg)
        xs_ref[0:N_NODES] = x_ref[...] * dsi[0:N_NODES, None]
        xs_ref[N_NODES:NPAD] = jnp.zeros((NPAD - N_NODES, D), jnp.float32)
        dsi_ref[...] = dsi[:, None]
        logits = jnp.dot(gf_ref[...], wg_ref[...], preferred_element_type=jnp.float32)
        logits = logits * (1.0 / TEMP)
        m = jnp.max(logits, axis=-1, keepdims=True)
        e = jnp.exp(logits - m)
        gate_ref[0:N_NODES] = e / jnp.sum(e, axis=-1, keepdims=True)
        gate_ref[N_NODES:NPAD] = jnp.zeros((NPAD - N_NODES, EXPERTS), jnp.float32)

    return pl.pallas_call(
        body,
        out_shape=(
            jax.ShapeDtypeStruct((NPAD, D), jnp.float32),
            jax.ShapeDtypeStruct((NPAD, 1), jnp.float32),
            jax.ShapeDtypeStruct((NPAD, EXPERTS), jnp.float32),
        ),
    )(deg_parts, x, gf, Wg)


def _combine_call(agg_parts, xs, dsi, gate, W, b):
    BR = 2000

    def body(a_ref, xs_ref, dsi_ref, gate_ref, w_ref, b_ref, o_ref):
        agg = (a_ref[0] + a_ref[1] + xs_ref[...]) * dsi_ref[...]
        acc = jnp.zeros((BR, D), jnp.float32)
        for i in range(EXPERTS):
            h = jnp.dot(agg, w_ref[i], preferred_element_type=jnp.float32)
            h = h + b_ref[i][None, :]
            acc = acc + gate_ref[:, i][:, None] * jnp.maximum(h, 0.0)
        o_ref[...] = acc

    return pl.pallas_call(
        body,
        grid=(N_NODES // BR,),
        in_specs=[
            pl.BlockSpec((NC, BR, D), lambda i: (0, i, 0)),
            pl.BlockSpec((BR, D), lambda i: (i, 0)),
            pl.BlockSpec((BR, 1), lambda i: (i, 0)),
            pl.BlockSpec((BR, EXPERTS), lambda i: (i, 0)),
            pl.BlockSpec((EXPERTS, D, D), lambda i: (0, 0, 0)),
            pl.BlockSpec((EXPERTS, D), lambda i: (0, 0)),
        ],
        out_specs=pl.BlockSpec((BR, D), lambda i: (i, 0)),
        out_shape=jax.ShapeDtypeStruct((N_NODES, D), jnp.float32),
    )(agg_parts, xs, dsi, gate, W, b)


def kernel(x, edge_index, gate_features, W, b, Wg):
    ei = edge_index.astype(jnp.int32)
    e = ei.shape[1]
    if e % CHUNK != 0:
        pad = CHUNK - e % CHUNK
        pad_rows = N_NODES + (jnp.arange(pad, dtype=jnp.int32) % (NPAD - N_NODES))
        ei = jnp.concatenate([ei, jnp.stack([pad_rows, pad_rows])], axis=1)
        e = e + pad
    base, extra, tail = _chunk_plan(e)

    deg_parts = _deg_call(ei, base, extra, tail)
    xs, dsi, gate = _prep_call(deg_parts, x, gate_features, Wg)
    agg_parts = _agg_call(xs, ei, base, extra, tail)
    return _combine_call(agg_parts, xs, dsi, gate, W, b)


# speedup vs baseline: 1.0937x; 1.0937x over previous
import functools

import jax
import jax.numpy as jnp
from jax import lax
from jax.experimental import pallas as pl
from jax.experimental.pallas import tpu as pltpu
from jax.experimental.pallas import tpu_sc as plsc

N_NODES = 10000
D = 128
EXPERTS = 3
TEMP = 101.0

NC = 2
NS = 16
NW = NC * NS
L = 16

CHUNK = 128
RING = 4
NPAD = 10240
RPT = NPAD // NS


def _mesh():
    return plsc.VectorSubcoreMesh(
        core_axis_name="c", subcore_axis_name="s", num_cores=NC, num_subcores=NS
    )


def _chunk_plan(e):
    nchunks = -(-e // CHUNK)
    base = nchunks // NW
    extra = nchunks % NW
    return base, extra, e % CHUNK


def _wid_start(wid, base, extra):
    return wid * base + jnp.minimum(wid, extra)


def _deg_call(ei, base, extra, tail):
    assert tail == 0, "deg kernel assumes CHUNK-aligned edge count"
    emax = (base + (1 if extra else 0)) * CHUNK

    @functools.partial(
        pl.kernel,
        mesh=_mesh(),
        out_type=jax.ShapeDtypeStruct((NW, NPAD), jnp.float32),
        scratch_types=[
            pltpu.VMEM((emax,), jnp.int32),
            pltpu.VMEM((NPAD,), jnp.float32),
        ],
        compiler_params=pltpu.CompilerParams(needs_layout_passes=False),
    )
    def k(ei_hbm, out_hbm, idxv, hist):
        cid = lax.axis_index("c")
        sid = lax.axis_index("s")
        wid = sid * NC + cid
        start = _wid_start(wid, base, extra) * CHUNK

        def fill_zero(i, carry):
            hist[pl.ds(i * L, L)] = jnp.zeros((L,), jnp.float32)
            return carry

        lax.fori_loop(0, NPAD // L, fill_zero, 0)
        pltpu.sync_copy(
            ei_hbm.at[1, pl.ds(start, base * CHUNK)],
            idxv.at[pl.ds(0, base * CHUNK)],
        )
        if extra:
            @pl.when(wid < extra)
            def _():
                pltpu.sync_copy(
                    ei_hbm.at[1, pl.ds(start + base * CHUNK, CHUNK)],
                    idxv.at[pl.ds(base * CHUNK, CHUNK)],
                )
        ones16 = jnp.ones((L,), jnp.float32)

        def body(i, carry):
            idx = idxv[pl.ds(i * L, L)]
            plsc.addupdate_scatter(hist, [idx], ones16)
            return carry

        lax.fori_loop(0, base * CHUNK // L, body, 0)
        if extra:
            @pl.when(wid < extra)
            def _():
                lax.fori_loop(
                    base * CHUNK // L, (base + 1) * CHUNK // L, body, 0
                )
        pltpu.sync_copy(hist, out_hbm.at[wid])

    return k(ei)


def _agg_call(xs, ei, base, extra, tail):
    assert tail == 0, "agg kernel assumes CHUNK-aligned edge count"
    pairs = base // 2
    odd = base % 2

    @functools.partial(
        pl.kernel,
        mesh=_mesh(),
        out_type=jax.ShapeDtypeStruct((NC, NPAD, D), jnp.float32),
        scratch_types=[
            pltpu.VMEM((RING, CHUNK), jnp.int32),
            pltpu.VMEM((RING, CHUNK), jnp.int32),
            pltpu.VMEM((CHUNK, D), jnp.float32),
            pltpu.VMEM((CHUNK, D), jnp.float32),
            pltpu.VMEM_SHARED((NPAD, D), jnp.float32),
            pltpu.SemaphoreType.DMA,
            pltpu.SemaphoreType.DMA,
            pltpu.SemaphoreType.DMA,
            pltpu.SemaphoreType.DMA,
        ],
    )
    def k(xs_hbm, ei_hbm, out_hbm, sxv, dxv, ra, rb, agg_sh, sie, sio, sa, sb):
        cid = lax.axis_index("c")
        sid = lax.axis_index("s")
        wid = sid * NC + cid
        start = _wid_start(wid, base, extra) * CHUNK
        n = base + jnp.where(wid < extra, 1, 0)

        def zfill(i, carry):
            rb[lax.div(i, D // L), pl.ds(lax.rem(i, D // L) * L, L)] = jnp.zeros(
                (L,), jnp.float32
            )
            return carry

        lax.fori_loop(0, CHUNK * (D // L), zfill, 0)
        nz = -(-RPT // CHUNK)
        for kk in range(nz):
            rows = min(CHUNK, RPT - kk * CHUNK)
            pltpu.sync_copy(
                rb.at[pl.ds(0, rows)],
                agg_sh.at[pl.ds(sid * RPT + kk * CHUNK, rows)],
            )

        def ipf(c, slot, sem):
            off = start + c * CHUNK
            pltpu.async_copy(ei_hbm.at[0, pl.ds(off, CHUNK)], sxv.at[slot], sem)
            pltpu.async_copy(ei_hbm.at[1, pl.ds(off, CHUNK)], dxv.at[slot], sem)

        def iwait(sem):
            pltpu.make_async_copy(ei_hbm.at[0, pl.ds(0, CHUNK)], sxv.at[0], sem).wait()
            pltpu.make_async_copy(ei_hbm.at[1, pl.ds(0, CHUNK)], dxv.at[0], sem).wait()

        def gwait(buf, sem):
            pltpu.make_async_copy(xs_hbm.at[sxv.at[0]], buf, sem).wait()

        ipf(0, 0, sie)
        iwait(sie)
        ipf(1, 1, sio)
        ipf(2, 2, sie)
        plsc.subcore_barrier()
        pltpu.async_copy(xs_hbm.at[sxv.at[0]], ra, sa)

        def body(i, carry):
            c = 2 * i
            r0 = lax.rem(c, RING)
            r1 = lax.rem(c + 1, RING)
            r2 = lax.rem(c + 2, RING)
            r3 = lax.rem(c + 3, RING)
            iwait(sio)
            pltpu.async_copy(xs_hbm.at[sxv.at[r1]], rb, sb)

            @pl.when(c + 3 < n)
            def _():
                ipf(c + 3, r3, sio)

            gwait(ra, sa)
            pltpu.sync_copy(ra, agg_sh.at[dxv.at[r0]], add=True)

            @pl.when(c + 2 < n)
            def _():
                iwait(sie)
                pltpu.async_copy(xs_hbm.at[sxv.at[r2]], ra, sa)

                @pl.when(c + 4 < n)
                def _():
                    ipf(c + 4, lax.rem(c + 4, RING), sie)

            gwait(rb, sb)
            pltpu.sync_copy(rb, agg_sh.at[dxv.at[r1]], add=True)
            return carry

        lax.fori_loop(0, pairs, body, 0)
        if odd:
            rz = (base - 1) % RING
            buf, sem = (ra, sa) if (base - 1) % 2 == 0 else (rb, sb)
            gwait(buf, sem)
            pltpu.sync_copy(buf, agg_sh.at[dxv.at[rz]], add=True)
        if extra:
            @pl.when(wid < extra)
            def _():
                rz = base % RING
                buf, sem = (ra, sa) if base % 2 == 0 else (rb, sb)
                gwait(buf, sem)
                pltpu.sync_copy(buf, agg_sh.at[dxv.at[rz]], add=True)
        plsc.subcore_barrier()
        pltpu.sync_copy(
            agg_sh.at[pl.ds(sid * RPT, RPT)],
            out_hbm.at[cid, pl.ds(sid * RPT, RPT)],
        )

    return k(xs, ei)


def _prep_call(deg_parts, x, gf, Wg):
    def body(degp_ref, x_ref, gf_ref, wg_ref, xs_ref, dsi_ref, gate_ref):
        deg = jnp.sum(degp_ref[...], axis=0) + 1.0
        dsi = lax.rsqrt(deg)
        xs_ref[0:N_NODES] = x_ref[...] * dsi[0:N_NODES, None]
        xs_ref[N_NODES:NPAD] = jnp.zeros((NPAD - N_NODES, D), jnp.float32)
        dsi_ref[...] = dsi[:, None]
        logits = jnp.dot(gf_ref[...], wg_ref[...], preferred_element_type=jnp.float32)
        logits = logits * (1.0 / TEMP)
        m = jnp.max(logits, axis=-1, keepdims=True)
        e = jnp.exp(logits - m)
        gate_ref[0:N_NODES] = e / jnp.sum(e, axis=-1, keepdims=True)
        gate_ref[N_NODES:NPAD] = jnp.zeros((NPAD - N_NODES, EXPERTS), jnp.float32)

    return pl.pallas_call(
        body,
        out_shape=(
            jax.ShapeDtypeStruct((NPAD, D), jnp.float32),
            jax.ShapeDtypeStruct((NPAD, 1), jnp.float32),
            jax.ShapeDtypeStruct((NPAD, EXPERTS), jnp.float32),
        ),
    )(deg_parts, x, gf, Wg)


def _combine_call(agg_parts, xs, dsi, gate, W, b):
    BR = 2000

    def body(a_ref, xs_ref, dsi_ref, gate_ref, w_ref, b_ref, o_ref):
        agg = (a_ref[0] + a_ref[1] + xs_ref[...]) * dsi_ref[...]
        acc = jnp.zeros((BR, D), jnp.float32)
        for i in range(EXPERTS):
            h = jnp.dot(agg, w_ref[i], preferred_element_type=jnp.float32)
            h = h + b_ref[i][None, :]
            acc = acc + gate_ref[:, i][:, None] * jnp.maximum(h, 0.0)
        o_ref[...] = acc

    return pl.pallas_call(
        body,
        grid=(N_NODES // BR,),
        in_specs=[
            pl.BlockSpec((NC, BR, D), lambda i: (0, i, 0)),
            pl.BlockSpec((BR, D), lambda i: (i, 0)),
            pl.BlockSpec((BR, 1), lambda i: (i, 0)),
            pl.BlockSpec((BR, EXPERTS), lambda i: (i, 0)),
            pl.BlockSpec((EXPERTS, D, D), lambda i: (0, 0, 0)),
            pl.BlockSpec((EXPERTS, D), lambda i: (0, 0)),
        ],
        out_specs=pl.BlockSpec((BR, D), lambda i: (i, 0)),
        out_shape=jax.ShapeDtypeStruct((N_NODES, D), jnp.float32),
    )(agg_parts, xs, dsi, gate, W, b)


def kernel(x, edge_index, gate_features, W, b, Wg):
    ei = edge_index.astype(jnp.int32)
    e = ei.shape[1]
    if e % CHUNK != 0:
        pad = CHUNK - e % CHUNK
        pad_rows = N_NODES + (jnp.arange(pad, dtype=jnp.int32) % (NPAD - N_NODES))
        ei = jnp.concatenate([ei, jnp.stack([pad_rows, pad_rows])], axis=1)
        e = e + pad
    base, extra, tail = _chunk_plan(e)

    deg_parts = _deg_call(ei, base, extra, tail)
    xs, dsi, gate = _prep_call(deg_parts, x, gate_features, Wg)
    agg_parts = _agg_call(xs, ei, base, extra, tail)
    return _combine_call(agg_parts, xs, dsi, gate, W, b)
